# baseline (speedup 1.0000x reference)
import functools

import jax
import jax.numpy as jnp
from jax import lax
from jax.experimental import pallas as pl
from jax.experimental.pallas import tpu as pltpu
from jax.experimental.pallas import tpu_sc as plsc

N_NODES = 10000
N_EDGES = 320000
D_FEAT = 128
HIDDEN = 64
NUM_GRAPHS = 64

NC = 2
NS = 16
NW = NC * NS
EPW = N_EDGES // NW
CHUNK = 80
NCHUNK = EPW // CHUNK
NPAD = 10240
ROWS_PER_TILE = NPAD // NS

_mesh = plsc.VectorSubcoreMesh(core_axis_name="c", subcore_axis_name="s")


@functools.partial(
    pl.kernel,
    out_type=jax.ShapeDtypeStruct((NC, NPAD, 16), jnp.float32),
    mesh=_mesh,
    scratch_types=[
        pltpu.VMEM((NCHUNK, CHUNK), jnp.int32),
        pltpu.VMEM((CHUNK, 16), jnp.float32),
        pltpu.VMEM_SHARED((NPAD, 16), jnp.float32),
    ],
    compiler_params=pltpu.CompilerParams(use_tc_tiling_on_sc=False),
)
def _deg_kernel(dsts_hbm, ones_hbm, zeros_hbm, out_hbm, dst_v, ones_v, acc_sh):
    c = lax.axis_index("c")
    s = lax.axis_index("s")
    wid = s * NC + c
    pltpu.sync_copy(dsts_hbm.at[wid], dst_v)
    pltpu.sync_copy(ones_hbm, ones_v)
    pltpu.sync_copy(zeros_hbm.at[pl.ds(s * ROWS_PER_TILE, ROWS_PER_TILE)],
                    acc_sh.at[pl.ds(s * ROWS_PER_TILE, ROWS_PER_TILE)])
    plsc.subcore_barrier()

    def body(j, carry):
        pltpu.sync_copy(ones_v, acc_sh.at[dst_v.at[j]], add=True)
        return carry

    lax.fori_loop(0, NCHUNK, body, 0)
    plsc.subcore_barrier()
    pltpu.sync_copy(acc_sh.at[pl.ds(s * ROWS_PER_TILE, ROWS_PER_TILE)],
                    out_hbm.at[c, pl.ds(s * ROWS_PER_TILE, ROWS_PER_TILE)])


NBUF = 5


@functools.partial(
    pl.kernel,
    out_type=jax.ShapeDtypeStruct((NC, NPAD, HIDDEN), jnp.float32),
    mesh=_mesh,
    scratch_types=[
        pltpu.VMEM((NCHUNK, CHUNK), jnp.int32),
        pltpu.VMEM((NCHUNK, CHUNK), jnp.int32),
        [pltpu.VMEM((CHUNK, HIDDEN), jnp.float32) for _ in range(NBUF)],
        pltpu.VMEM_SHARED((NPAD, HIDDEN), jnp.float32),
        pltpu.VMEM_SHARED((NPAD, HIDDEN), jnp.float32),
        [pltpu.SemaphoreType.DMA for _ in range(NBUF)],
    ],
    compiler_params=pltpu.CompilerParams(use_tc_tiling_on_sc=False),
)
def _agg_kernel(t_hbm, srcs_hbm, dsts_hbm, zeros_hbm, out_hbm,
                src_v, dst_v, rows_v, acc_sh, t_sh, sems):
    c = lax.axis_index("c")
    s = lax.axis_index("s")
    wid = s * NC + c
    pltpu.sync_copy(srcs_hbm.at[wid], src_v)
    pltpu.sync_copy(dsts_hbm.at[wid], dst_v)
    pltpu.sync_copy(zeros_hbm.at[pl.ds(s * ROWS_PER_TILE, ROWS_PER_TILE)],
                    acc_sh.at[pl.ds(s * ROWS_PER_TILE, ROWS_PER_TILE)])
    pltpu.sync_copy(t_hbm.at[pl.ds(s * ROWS_PER_TILE, ROWS_PER_TILE)],
                    t_sh.at[pl.ds(s * ROWS_PER_TILE, ROWS_PER_TILE)])
    plsc.subcore_barrier()

    for b in range(NBUF):
        pltpu.async_copy(t_sh.at[src_v.at[b]], rows_v[b], sems[b])

    def body(g, carry):
        base = g * NBUF
        for b in range(NBUF):
            j = base + b
            pltpu.make_async_copy(t_sh.at[src_v.at[j]], rows_v[b],
                                  sems[b]).wait()
            pltpu.sync_copy(rows_v[b], acc_sh.at[dst_v.at[j]], add=True)

            @pl.when(j + NBUF < NCHUNK)
            def _():
                pltpu.async_copy(t_sh.at[src_v.at[j + NBUF]], rows_v[b],
                                 sems[b])
        return carry

    lax.fori_loop(0, NCHUNK // NBUF, body, 0)
    plsc.subcore_barrier()
    pltpu.sync_copy(acc_sh.at[pl.ds(s * ROWS_PER_TILE, ROWS_PER_TILE)],
                    out_hbm.at[c, pl.ds(s * ROWS_PER_TILE, ROWS_PER_TILE)])


def _dinv(degp_ref):
    deg = degp_ref[0, 0:N_NODES, 0:1] + degp_ref[1, 0:N_NODES, 0:1] + 1.0
    return 1.0 / jnp.sqrt(deg)


def _dense1_body(x_ref, w1_ref, degp_ref, t1_ref):
    dinv = _dinv(degp_ref)
    xw = jnp.dot(x_ref[...], w1_ref[...], preferred_element_type=jnp.float32)
    t1_ref[0:N_NODES, :] = xw * dinv
    t1_ref[N_NODES:NPAD, :] = jnp.zeros((NPAD - N_NODES, HIDDEN), jnp.float32)


def _dense2_body(aggp_ref, t1_ref, degp_ref, b1_ref, w2_ref, t2_ref):
    dinv = _dinv(degp_ref)
    agg = (aggp_ref[0, 0:N_NODES, :] + aggp_ref[1, 0:N_NODES, :]
           + t1_ref[0:N_NODES, :])
    h1 = jnp.maximum(agg * dinv + b1_ref[...], 0.0)
    t2_ref[0:N_NODES, :] = jnp.dot(h1, w2_ref[...],
                                   preferred_element_type=jnp.float32) * dinv
    t2_ref[N_NODES:NPAD, :] = jnp.zeros((NPAD - N_NODES, HIDDEN), jnp.float32)


def _dense3_body(aggp_ref, t2_ref, degp_ref, b2_ref, batch_ref, fcw_ref,
                 fcb_ref, out_ref):
    dinv = _dinv(degp_ref)
    agg = (aggp_ref[0, 0:N_NODES, :] + aggp_ref[1, 0:N_NODES, :]
           + t2_ref[0:N_NODES, :])
    h2 = jnp.maximum(agg * dinv + b2_ref[...], 0.0)
    gids = lax.broadcasted_iota(jnp.int32, (NUM_GRAPHS, N_NODES), 0)
    onehot = (batch_ref[...] == gids).astype(jnp.float32)
    sums = jnp.dot(onehot, h2, preferred_element_type=jnp.float32,
                   precision=lax.Precision.HIGHEST)
    counts = jnp.sum(onehot, axis=1, keepdims=True)
    pooled = sums / jnp.maximum(counts, 1.0)
    out_ref[...] = jnp.dot(pooled, fcw_ref[...],
                           preferred_element_type=jnp.float32) + fcb_ref[...]


def _tc_call(body, out_shape, *args):
    return pl.pallas_call(
        body, out_shape=jax.ShapeDtypeStruct(out_shape, jnp.float32))(*args)


def kernel(x, edge_index, batch, W1, b1, W2, b2, fc_w, fc_b):
    src = edge_index[0].astype(jnp.int32).reshape(NW, NCHUNK, CHUNK)
    dst = edge_index[1].astype(jnp.int32).reshape(NW, NCHUNK, CHUNK)
    batch2d = batch.astype(jnp.int32).reshape(1, N_NODES)

    ones16 = jnp.ones((CHUNK, 16), jnp.float32)
    zeros16 = jnp.zeros((NPAD, 16), jnp.float32)
    zeros64 = jnp.zeros((NPAD, HIDDEN), jnp.float32)

    degp = _deg_kernel(dst, ones16, zeros16)

    t1 = _tc_call(_dense1_body, (NPAD, HIDDEN), x, W1, degp)
    agg1 = _agg_kernel(t1, src, dst, zeros64)
    t2 = _tc_call(_dense2_body, (NPAD, HIDDEN), agg1, t1, degp,
                  b1.reshape(1, HIDDEN), W2)
    agg2 = _agg_kernel(t2, src, dst, zeros64)
    out = _tc_call(_dense3_body, (NUM_GRAPHS, 1), agg2, t2, degp,
                   b2.reshape(1, HIDDEN), batch2d, fc_w,
                   fc_b.reshape(1, 1))
    return out


# speedup vs baseline: 1.2211x; 1.2211x over previous
import functools

import jax
import jax.numpy as jnp
from jax import lax
from jax.experimental import pallas as pl
from jax.experimental.pallas import tpu as pltpu
from jax.experimental.pallas import tpu_sc as plsc

N_NODES = 10000
N_EDGES = 320000
D_FEAT = 128
HIDDEN = 64
NUM_GRAPHS = 64

NC = 2
NS = 16
NW = NC * NS
EPW = N_EDGES // NW
CHUNK = 80
NCHUNK = EPW // CHUNK
NPAD = 10240
ROWS_PER_TILE = NPAD // NS

_mesh = plsc.VectorSubcoreMesh(core_axis_name="c", subcore_axis_name="s")


@functools.partial(
    pl.kernel,
    out_type=jax.ShapeDtypeStruct((NC, NPAD, 16), jnp.float32),
    mesh=_mesh,
    scratch_types=[
        pltpu.VMEM((NCHUNK, CHUNK), jnp.int32),
        pltpu.VMEM((CHUNK, 16), jnp.float32),
        pltpu.VMEM_SHARED((NPAD, 16), jnp.float32),
    ],
    compiler_params=pltpu.CompilerParams(use_tc_tiling_on_sc=False),
)
def _deg_kernel(dsts_hbm, ones_hbm, zeros_hbm, out_hbm, dst_v, ones_v, acc_sh):
    c = lax.axis_index("c")
    s = lax.axis_index("s")
    wid = s * NC + c
    pltpu.sync_copy(dsts_hbm.at[wid], dst_v)
    pltpu.sync_copy(ones_hbm, ones_v)
    pltpu.sync_copy(zeros_hbm.at[pl.ds(s * ROWS_PER_TILE, ROWS_PER_TILE)],
                    acc_sh.at[pl.ds(s * ROWS_PER_TILE, ROWS_PER_TILE)])
    plsc.subcore_barrier()

    def body(j, carry):
        pltpu.sync_copy(ones_v, acc_sh.at[dst_v.at[j]], add=True)
        return carry

    lax.fori_loop(0, NCHUNK, body, 0)
    plsc.subcore_barrier()
    pltpu.sync_copy(acc_sh.at[pl.ds(s * ROWS_PER_TILE, ROWS_PER_TILE)],
                    out_hbm.at[c, pl.ds(s * ROWS_PER_TILE, ROWS_PER_TILE)])


NBUF = 5


@functools.partial(
    pl.kernel,
    out_type=jax.ShapeDtypeStruct((NC, NPAD, HIDDEN), jnp.float32),
    mesh=_mesh,
    scratch_types=[
        pltpu.VMEM((NCHUNK, CHUNK), jnp.int32),
        pltpu.VMEM((NCHUNK, CHUNK), jnp.int32),
        [pltpu.VMEM((CHUNK, HIDDEN), jnp.float32) for _ in range(NBUF)],
        pltpu.VMEM_SHARED((NPAD, HIDDEN), jnp.float32),
        [pltpu.SemaphoreType.DMA for _ in range(NBUF)],
    ],
    compiler_params=pltpu.CompilerParams(use_tc_tiling_on_sc=False),
)
def _agg_kernel(t_hbm, srcs_hbm, dsts_hbm, zeros_hbm, out_hbm,
                src_v, dst_v, rows_v, acc_sh, sems):
    c = lax.axis_index("c")
    s = lax.axis_index("s")
    wid = s * NC + c
    pltpu.sync_copy(srcs_hbm.at[wid], src_v)
    pltpu.sync_copy(dsts_hbm.at[wid], dst_v)
    pltpu.sync_copy(zeros_hbm.at[pl.ds(s * ROWS_PER_TILE, ROWS_PER_TILE)],
                    acc_sh.at[pl.ds(s * ROWS_PER_TILE, ROWS_PER_TILE)])
    plsc.subcore_barrier()

    for b in range(NBUF):
        pltpu.async_copy(t_hbm.at[src_v.at[b]], rows_v[b], sems[b])

    def body(g, carry):
        base = g * NBUF
        for b in range(NBUF):
            j = base + b
            pltpu.make_async_copy(t_hbm.at[src_v.at[j]], rows_v[b],
                                  sems[b]).wait()
            pltpu.sync_copy(rows_v[b], acc_sh.at[dst_v.at[j]], add=True)

            @pl.when(j + NBUF < NCHUNK)
            def _():
                pltpu.async_copy(t_hbm.at[src_v.at[j + NBUF]], rows_v[b],
                                 sems[b])
        return carry

    lax.fori_loop(0, NCHUNK // NBUF, body, 0)
    plsc.subcore_barrier()
    pltpu.sync_copy(acc_sh.at[pl.ds(s * ROWS_PER_TILE, ROWS_PER_TILE)],
                    out_hbm.at[c, pl.ds(s * ROWS_PER_TILE, ROWS_PER_TILE)])


BR = 1280
NBLK = NPAD // BR


def _dinv_blk(degp_ref):
    deg = degp_ref[0, :, 0:1] + degp_ref[1, :, 0:1] + 1.0
    return 1.0 / jnp.sqrt(deg)


def _dense1_body(x_ref, w1_ref, degp_ref, t1_ref):
    dinv = _dinv_blk(degp_ref)
    xw = jnp.dot(x_ref[...], w1_ref[...], preferred_element_type=jnp.float32)
    t1_ref[...] = xw * dinv


def _dense2_body(aggp_ref, t1_ref, degp_ref, b1_ref, w2_ref, t2_ref):
    dinv = _dinv_blk(degp_ref)
    agg = aggp_ref[0] + aggp_ref[1] + t1_ref[...]
    h1 = jnp.maximum(agg * dinv + b1_ref[...], 0.0)
    t2_ref[...] = jnp.dot(h1, w2_ref[...],
                          preferred_element_type=jnp.float32) * dinv


def _dense3_body(aggp_ref, t2_ref, degp_ref, b2_ref, batch_ref, fcw_ref,
                 fcb_ref, out_ref, sums_ref, counts_ref):
    i = pl.program_id(0)
    dinv = _dinv_blk(degp_ref)
    agg = aggp_ref[0] + aggp_ref[1] + t2_ref[...]
    h2 = jnp.maximum(agg * dinv + b2_ref[...], 0.0)
    rowid = lax.broadcasted_iota(jnp.int32, (BR, 1), 0) + i * BR
    h2 = jnp.where(rowid < N_NODES, h2, 0.0)
    gids = lax.broadcasted_iota(jnp.int32, (NUM_GRAPHS, BR), 0)
    colid = lax.broadcasted_iota(jnp.int32, (NUM_GRAPHS, BR), 1) + i * BR
    onehot = ((batch_ref[...] == gids) & (colid < N_NODES)).astype(jnp.float32)
    part = jnp.dot(onehot, h2, preferred_element_type=jnp.float32,
                   precision=lax.Precision.HIGHEST)
    pcnt = jnp.sum(onehot, axis=1, keepdims=True)

    @pl.when(i == 0)
    def _():
        sums_ref[...] = part
        counts_ref[...] = pcnt

    @pl.when(i > 0)
    def _():
        sums_ref[...] += part
        counts_ref[...] += pcnt

    @pl.when(i == NBLK - 1)
    def _():
        pooled = sums_ref[...] / jnp.maximum(counts_ref[...], 1.0)
        out_ref[...] = jnp.dot(pooled, fcw_ref[...],
                               preferred_element_type=jnp.float32) \
            + fcb_ref[...]


def _row_spec(width):
    return pl.BlockSpec((BR, width), lambda i: (i, 0))


def _full_spec(shape):
    nd = len(shape)
    return pl.BlockSpec(shape, lambda i, _nd=nd: (0,) * _nd)


_DEGP_SPEC = pl.BlockSpec((NC, BR, 16), lambda i: (0, i, 0))
_AGGP_SPEC = pl.BlockSpec((NC, BR, HIDDEN), lambda i: (0, i, 0))


def _dense1(x, W1, degp):
    return pl.pallas_call(
        _dense1_body,
        grid=(NBLK,),
        in_specs=[_row_spec(D_FEAT), _full_spec((D_FEAT, HIDDEN)),
                  _DEGP_SPEC],
        out_specs=_row_spec(HIDDEN),
        out_shape=jax.ShapeDtypeStruct((NPAD, HIDDEN), jnp.float32),
    )(x, W1, degp)


def _dense2(aggp, t1, degp, b1, W2):
    return pl.pallas_call(
        _dense2_body,
        grid=(NBLK,),
        in_specs=[_AGGP_SPEC, _row_spec(HIDDEN), _DEGP_SPEC,
                  _full_spec((1, HIDDEN)), _full_spec((HIDDEN, HIDDEN))],
        out_specs=_row_spec(HIDDEN),
        out_shape=jax.ShapeDtypeStruct((NPAD, HIDDEN), jnp.float32),
    )(aggp, t1, degp, b1, W2)


def _dense3(aggp, t2, degp, b2, batch2d, fc_w, fc_b):
    return pl.pallas_call(
        _dense3_body,
        grid=(NBLK,),
        in_specs=[_AGGP_SPEC, _row_spec(HIDDEN), _DEGP_SPEC,
                  _full_spec((1, HIDDEN)),
                  pl.BlockSpec((1, BR), lambda i: (0, i)),
                  _full_spec((HIDDEN, 1)), _full_spec((1, 1))],
        out_specs=_full_spec((NUM_GRAPHS, 1)),
        out_shape=jax.ShapeDtypeStruct((NUM_GRAPHS, 1), jnp.float32),
        scratch_shapes=[pltpu.VMEM((NUM_GRAPHS, HIDDEN), jnp.float32),
                        pltpu.VMEM((NUM_GRAPHS, 1), jnp.float32)],
    )(aggp, t2, degp, b2, batch2d, fc_w, fc_b)


def kernel(x, edge_index, batch, W1, b1, W2, b2, fc_w, fc_b):
    src = edge_index[0].astype(jnp.int32).reshape(NW, NCHUNK, CHUNK)
    dst = edge_index[1].astype(jnp.int32).reshape(NW, NCHUNK, CHUNK)
    batch2d = batch.astype(jnp.int32).reshape(1, N_NODES)

    ones16 = jnp.ones((CHUNK, 16), jnp.float32)
    zeros16 = jnp.zeros((NPAD, 16), jnp.float32)
    zeros64 = jnp.zeros((NPAD, HIDDEN), jnp.float32)

    degp = _deg_kernel(dst, ones16, zeros16)

    t1 = _dense1(x, W1, degp)
    agg1 = _agg_kernel(t1, src, dst, zeros64)
    t2 = _dense2(agg1, t1, degp, b1.reshape(1, HIDDEN), W2)
    agg2 = _agg_kernel(t2, src, dst, zeros64)
    out = _dense3(agg2, t2, degp, b2.reshape(1, HIDDEN), batch2d, fc_w,
                  fc_b.reshape(1, 1))
    return out


# speedup vs baseline: 1.4113x; 1.1557x over previous
import functools

import jax
import jax.numpy as jnp
from jax import lax
from jax.experimental import pallas as pl
from jax.experimental.pallas import tpu as pltpu
from jax.experimental.pallas import tpu_sc as plsc

N_NODES = 10000
N_EDGES = 320000
D_FEAT = 128
HIDDEN = 64
NUM_GRAPHS = 64

NC = 2
NS = 16
NW = NC * NS
EPW = N_EDGES // NW
CHUNK = 80
NCHUNK = EPW // CHUNK
NPAD = 10240
ROWS_PER_TILE = NPAD // NS

_mesh = plsc.VectorSubcoreMesh(core_axis_name="c", subcore_axis_name="s")


@functools.partial(
    pl.kernel,
    out_type=jax.ShapeDtypeStruct((NC, NPAD, 16), jnp.float32),
    mesh=_mesh,
    scratch_types=[
        pltpu.VMEM((NCHUNK, CHUNK), jnp.int32),
        pltpu.VMEM((CHUNK, 16), jnp.float32),
        pltpu.VMEM_SHARED((NPAD, 16), jnp.float32),
    ],
    compiler_params=pltpu.CompilerParams(use_tc_tiling_on_sc=False),
)
def _deg_kernel(dsts_hbm, ones_hbm, zeros_hbm, out_hbm, dst_v, ones_v, acc_sh):
    c = lax.axis_index("c")
    s = lax.axis_index("s")
    wid = s * NC + c
    pltpu.sync_copy(dsts_hbm.at[wid], dst_v)
    pltpu.sync_copy(ones_hbm, ones_v)
    pltpu.sync_copy(zeros_hbm.at[pl.ds(s * ROWS_PER_TILE, ROWS_PER_TILE)],
                    acc_sh.at[pl.ds(s * ROWS_PER_TILE, ROWS_PER_TILE)])
    plsc.subcore_barrier()

    def body(j, carry):
        pltpu.sync_copy(ones_v, acc_sh.at[dst_v.at[j]], add=True)
        return carry

    lax.fori_loop(0, NCHUNK, body, 0)
    plsc.subcore_barrier()
    pltpu.sync_copy(acc_sh.at[pl.ds(s * ROWS_PER_TILE, ROWS_PER_TILE)],
                    out_hbm.at[c, pl.ds(s * ROWS_PER_TILE, ROWS_PER_TILE)])


NBUF = 5


@functools.partial(
    pl.kernel,
    out_type=jax.ShapeDtypeStruct((NC, NPAD, HIDDEN), jnp.float32),
    mesh=_mesh,
    scratch_types=[
        pltpu.VMEM((NCHUNK, CHUNK), jnp.int32),
        pltpu.VMEM((NCHUNK, CHUNK), jnp.int32),
        [pltpu.VMEM((CHUNK, HIDDEN), jnp.float32) for _ in range(NBUF)],
        pltpu.VMEM_SHARED((NPAD, HIDDEN), jnp.float32),
        [pltpu.SemaphoreType.DMA for _ in range(NBUF)],
    ],
    compiler_params=pltpu.CompilerParams(use_tc_tiling_on_sc=False),
)
def _agg_kernel(t_hbm, srcs_hbm, dsts_hbm, zeros_hbm, out_hbm,
                src_v, dst_v, rows_v, acc_sh, sems):
    c = lax.axis_index("c")
    s = lax.axis_index("s")
    wid = s * NC + c
    pltpu.sync_copy(srcs_hbm.at[wid], src_v)
    pltpu.sync_copy(dsts_hbm.at[wid], dst_v)
    pltpu.sync_copy(zeros_hbm.at[pl.ds(s * ROWS_PER_TILE, ROWS_PER_TILE)],
                    acc_sh.at[pl.ds(s * ROWS_PER_TILE, ROWS_PER_TILE)])
    plsc.subcore_barrier()

    for b in range(NBUF):
        pltpu.async_copy(t_hbm.at[src_v.at[b]], rows_v[b], sems[b])

    def body(g, carry):
        base = g * NBUF
        for b in range(NBUF):
            j = base + b
            pltpu.make_async_copy(t_hbm.at[src_v.at[j]], rows_v[b],
                                  sems[b]).wait()
            pltpu.sync_copy(rows_v[b], acc_sh.at[dst_v.at[j]], add=True)

            @pl.when(j + NBUF < NCHUNK)
            def _():
                pltpu.async_copy(t_hbm.at[src_v.at[j + NBUF]], rows_v[b],
                                 sems[b])
        return carry

    lax.fori_loop(0, NCHUNK // NBUF, body, 0)
    plsc.subcore_barrier()
    pltpu.sync_copy(acc_sh.at[pl.ds(s * ROWS_PER_TILE, ROWS_PER_TILE)],
                    out_hbm.at[c, pl.ds(s * ROWS_PER_TILE, ROWS_PER_TILE)])


BR = 1280
NBLK = NPAD // BR


def _pack(v):
    return jnp.concatenate([v[0:BR // 2], v[BR // 2:BR]], axis=1)


def _unpack(p):
    return jnp.concatenate([p[:, 0:HIDDEN], p[:, HIDDEN:2 * HIDDEN]], axis=0)


def _dinv_blk(degp_ref):
    deg_a = degp_ref[0, :, 0:1] + degp_ref[1, :, 0:1] + 1.0
    deg_b = degp_ref[0, :, 16:17] + degp_ref[1, :, 16:17] + 1.0
    return 1.0 / jnp.sqrt(jnp.concatenate([deg_a, deg_b], axis=0))


def _dense1_body(x_ref, w1_ref, degp_ref, t1_ref):
    dinv = _dinv_blk(degp_ref)
    xw = jnp.dot(x_ref[...], w1_ref[...], preferred_element_type=jnp.float32)
    t1_ref[...] = _pack(xw * dinv)


def _dense2_body(aggp_ref, t1_ref, degp_ref, b1_ref, w2_ref, t2_ref):
    dinv = _dinv_blk(degp_ref)
    agg = _unpack(aggp_ref[0] + aggp_ref[1] + t1_ref[...])
    h1 = jnp.maximum(agg * dinv + b1_ref[...], 0.0)
    t2 = jnp.dot(h1, w2_ref[...], preferred_element_type=jnp.float32) * dinv
    t2_ref[...] = _pack(t2)


def _dense3_body(aggp_ref, t2_ref, degp_ref, b2_ref, batch_ref, fcw_ref,
                 fcb_ref, out_ref, sums_ref, counts_ref):
    i = pl.program_id(0)
    dinv = _dinv_blk(degp_ref)
    agg = _unpack(aggp_ref[0] + aggp_ref[1] + t2_ref[...])
    h2 = jnp.maximum(agg * dinv + b2_ref[...], 0.0)
    rowid = lax.broadcasted_iota(jnp.int32, (BR, 1), 0) + i * BR
    h2 = jnp.where(rowid < N_NODES, h2, 0.0)
    gids = lax.broadcasted_iota(jnp.int32, (NUM_GRAPHS, BR), 0)
    colid = lax.broadcasted_iota(jnp.int32, (NUM_GRAPHS, BR), 1) + i * BR
    onehot = ((batch_ref[...] == gids) & (colid < N_NODES)).astype(jnp.float32)
    part = jnp.dot(onehot, h2, preferred_element_type=jnp.float32,
                   precision=lax.Precision.HIGHEST)
    pcnt = jnp.sum(onehot, axis=1, keepdims=True)

    @pl.when(i == 0)
    def _():
        sums_ref[...] = part
        counts_ref[...] = pcnt

    @pl.when(i > 0)
    def _():
        sums_ref[...] += part
        counts_ref[...] += pcnt

    @pl.when(i == NBLK - 1)
    def _():
        pooled = sums_ref[...] / jnp.maximum(counts_ref[...], 1.0)
        out_ref[...] = jnp.dot(pooled, fcw_ref[...],
                               preferred_element_type=jnp.float32) \
            + fcb_ref[...]


def _row_spec(width):
    return pl.BlockSpec((BR, width), lambda i: (i, 0))


def _full_spec(shape):
    nd = len(shape)
    return pl.BlockSpec(shape, lambda i, _nd=nd: (0,) * _nd)


_DEGP_SPEC = pl.BlockSpec((NC, BR // 2, 32), lambda i: (0, i, 0))
_AGGP_SPEC = pl.BlockSpec((NC, BR // 2, 2 * HIDDEN), lambda i: (0, i, 0))
_PACK_SPEC = pl.BlockSpec((BR // 2, 2 * HIDDEN), lambda i: (i, 0))


def _dense1(x, W1, degp):
    return pl.pallas_call(
        _dense1_body,
        grid=(NBLK,),
        in_specs=[_row_spec(D_FEAT), _full_spec((D_FEAT, HIDDEN)),
                  _DEGP_SPEC],
        out_specs=_PACK_SPEC,
        out_shape=jax.ShapeDtypeStruct((NPAD // 2, 2 * HIDDEN), jnp.float32),
    )(x, W1, degp)


def _dense2(aggp, t1, degp, b1, W2):
    return pl.pallas_call(
        _dense2_body,
        grid=(NBLK,),
        in_specs=[_AGGP_SPEC, _PACK_SPEC, _DEGP_SPEC,
                  _full_spec((1, HIDDEN)), _full_spec((HIDDEN, HIDDEN))],
        out_specs=_PACK_SPEC,
        out_shape=jax.ShapeDtypeStruct((NPAD // 2, 2 * HIDDEN), jnp.float32),
    )(aggp, t1, degp, b1, W2)


def _dense3(aggp, t2, degp, b2, batch2d, fc_w, fc_b):
    return pl.pallas_call(
        _dense3_body,
        grid=(NBLK,),
        in_specs=[_AGGP_SPEC, _PACK_SPEC, _DEGP_SPEC,
                  _full_spec((1, HIDDEN)),
                  pl.BlockSpec((1, BR), lambda i: (0, i)),
                  _full_spec((HIDDEN, 1)), _full_spec((1, 1))],
        out_specs=_full_spec((NUM_GRAPHS, 1)),
        out_shape=jax.ShapeDtypeStruct((NUM_GRAPHS, 1), jnp.float32),
        scratch_shapes=[pltpu.VMEM((NUM_GRAPHS, HIDDEN), jnp.float32),
                        pltpu.VMEM((NUM_GRAPHS, 1), jnp.float32)],
    )(aggp, t2, degp, b2, batch2d, fc_w, fc_b)


def _pos(v):
    rem = v % BR
    return (v - rem) + 2 * (rem % (BR // 2)) + rem // (BR // 2)


def kernel(x, edge_index, batch, W1, b1, W2, b2, fc_w, fc_b):
    src = _pos(edge_index[0].astype(jnp.int32)).reshape(NW, NCHUNK, CHUNK)
    dst = _pos(edge_index[1].astype(jnp.int32)).reshape(NW, NCHUNK, CHUNK)
    batch2d = batch.astype(jnp.int32).reshape(1, N_NODES)

    ones16 = jnp.ones((CHUNK, 16), jnp.float32)
    zeros16 = jnp.zeros((NPAD, 16), jnp.float32)
    zeros64 = jnp.zeros((NPAD, HIDDEN), jnp.float32)

    degp = _deg_kernel(dst, ones16, zeros16)
    degp32 = degp.reshape(NC, NPAD // 2, 32)

    t1p = _dense1(x, W1, degp32)
    agg1 = _agg_kernel(t1p.reshape(NPAD, HIDDEN), src, dst, zeros64)
    t2p = _dense2(agg1.reshape(NC, NPAD // 2, 2 * HIDDEN), t1p, degp32,
                  b1.reshape(1, HIDDEN), W2)
    agg2 = _agg_kernel(t2p.reshape(NPAD, HIDDEN), src, dst, zeros64)
    out = _dense3(agg2.reshape(NC, NPAD // 2, 2 * HIDDEN), t2p, degp32,
                  b2.reshape(1, HIDDEN), batch2d, fc_w, fc_b.reshape(1, 1))
    return out


# speedup vs baseline: 1.4596x; 1.0342x over previous
import functools

import jax
import jax.numpy as jnp
from jax import lax
from jax.experimental import pallas as pl
from jax.experimental.pallas import tpu as pltpu
from jax.experimental.pallas import tpu_sc as plsc

N_NODES = 10000
N_EDGES = 320000
D_FEAT = 128
HIDDEN = 64
NUM_GRAPHS = 64

NC = 2
NS = 16
NW = NC * NS
EPW = N_EDGES // NW
CHUNK = 80
NCHUNK = EPW // CHUNK
NPAD = 10240
ROWS_PER_TILE = NPAD // NS

_mesh = plsc.VectorSubcoreMesh(core_axis_name="c", subcore_axis_name="s")


@functools.partial(
    pl.kernel,
    out_type=jax.ShapeDtypeStruct((NC, NPAD, 16), jnp.float32),
    mesh=_mesh,
    scratch_types=[
        pltpu.VMEM((NCHUNK, CHUNK), jnp.int32),
        pltpu.VMEM((CHUNK, 16), jnp.float32),
        pltpu.VMEM_SHARED((NPAD, 16), jnp.float32),
        pltpu.SemaphoreType.DMA,
    ],
    compiler_params=pltpu.CompilerParams(use_tc_tiling_on_sc=False),
)
def _deg_kernel(dsts_hbm, ones_hbm, zeros_hbm, out_hbm, dst_v, ones_v, acc_sh,
                dsem):
    c = lax.axis_index("c")
    s = lax.axis_index("s")
    wid = s * NC + c
    pltpu.sync_copy(dsts_hbm.at[wid], dst_v)
    pltpu.sync_copy(ones_hbm, ones_v)
    pltpu.sync_copy(zeros_hbm.at[pl.ds(s * ROWS_PER_TILE, ROWS_PER_TILE)],
                    acc_sh.at[pl.ds(s * ROWS_PER_TILE, ROWS_PER_TILE)])
    plsc.subcore_barrier()

    def body(j, carry):
        pltpu.async_copy(ones_v, acc_sh.at[dst_v.at[j]], dsem, add=True)
        return carry

    lax.fori_loop(0, NCHUNK, body, 0)

    def drain(j, carry):
        pltpu.make_async_copy(ones_v, acc_sh.at[dst_v.at[0]], dsem).wait()
        return carry

    lax.fori_loop(0, NCHUNK, drain, 0)
    plsc.subcore_barrier()
    pltpu.sync_copy(acc_sh.at[pl.ds(s * ROWS_PER_TILE, ROWS_PER_TILE)],
                    out_hbm.at[c, pl.ds(s * ROWS_PER_TILE, ROWS_PER_TILE)])


NBUF = 5


@functools.partial(
    pl.kernel,
    out_type=jax.ShapeDtypeStruct((NC, NPAD, HIDDEN), jnp.float32),
    mesh=_mesh,
    scratch_types=[
        pltpu.VMEM((NCHUNK, CHUNK), jnp.int32),
        pltpu.VMEM((NCHUNK, CHUNK), jnp.int32),
        [pltpu.VMEM((CHUNK, HIDDEN), jnp.float32) for _ in range(NBUF)],
        pltpu.VMEM_SHARED((NPAD, HIDDEN), jnp.float32),
        [pltpu.SemaphoreType.DMA for _ in range(NBUF)],
    ],
    compiler_params=pltpu.CompilerParams(use_tc_tiling_on_sc=False),
)
def _agg_kernel(t_hbm, srcs_hbm, dsts_hbm, zeros_hbm, out_hbm,
                src_v, dst_v, rows_v, acc_sh, sems):
    c = lax.axis_index("c")
    s = lax.axis_index("s")
    wid = s * NC + c
    pltpu.sync_copy(srcs_hbm.at[wid], src_v)
    pltpu.sync_copy(dsts_hbm.at[wid], dst_v)
    pltpu.sync_copy(zeros_hbm.at[pl.ds(s * ROWS_PER_TILE, ROWS_PER_TILE)],
                    acc_sh.at[pl.ds(s * ROWS_PER_TILE, ROWS_PER_TILE)])
    plsc.subcore_barrier()

    for b in range(NBUF):
        pltpu.async_copy(t_hbm.at[src_v.at[b]], rows_v[b], sems[b])

    def body(g, carry):
        base = g * NBUF
        for b in range(NBUF):
            j = base + b
            pltpu.make_async_copy(t_hbm.at[src_v.at[j]], rows_v[b],
                                  sems[b]).wait()
            pltpu.sync_copy(rows_v[b], acc_sh.at[dst_v.at[j]], add=True)

            @pl.when(j + NBUF < NCHUNK)
            def _():
                pltpu.async_copy(t_hbm.at[src_v.at[j + NBUF]], rows_v[b],
                                 sems[b])
        return carry

    lax.fori_loop(0, NCHUNK // NBUF, body, 0)
    plsc.subcore_barrier()
    pltpu.sync_copy(acc_sh.at[pl.ds(s * ROWS_PER_TILE, ROWS_PER_TILE)],
                    out_hbm.at[c, pl.ds(s * ROWS_PER_TILE, ROWS_PER_TILE)])


BR = 1280
NBLK = NPAD // BR


def _pack(v):
    return jnp.concatenate([v[0:BR // 2], v[BR // 2:BR]], axis=1)


def _unpack(p):
    return jnp.concatenate([p[:, 0:HIDDEN], p[:, HIDDEN:2 * HIDDEN]], axis=0)


def _dinv_blk(degp_ref):
    deg_a = degp_ref[0, :, 0:1] + degp_ref[1, :, 0:1] + 1.0
    deg_b = degp_ref[0, :, 16:17] + degp_ref[1, :, 16:17] + 1.0
    return 1.0 / jnp.sqrt(jnp.concatenate([deg_a, deg_b], axis=0))


def _dense1_body(x_ref, w1_ref, degp_ref, t1_ref):
    dinv = _dinv_blk(degp_ref)
    xw = jnp.dot(x_ref[...], w1_ref[...], preferred_element_type=jnp.float32)
    t1_ref[...] = _pack(xw * dinv)


def _dense2_body(aggp_ref, t1_ref, degp_ref, b1_ref, w2_ref, t2_ref):
    dinv = _dinv_blk(degp_ref)
    agg = _unpack(aggp_ref[0] + aggp_ref[1] + t1_ref[...])
    h1 = jnp.maximum(agg * dinv + b1_ref[...], 0.0)
    t2 = jnp.dot(h1, w2_ref[...], preferred_element_type=jnp.float32) * dinv
    t2_ref[...] = _pack(t2)


def _dense3_body(aggp_ref, t2_ref, degp_ref, b2_ref, batch_ref, fcw_ref,
                 fcb_ref, out_ref, sums_ref, counts_ref):
    i = pl.program_id(0)
    dinv = _dinv_blk(degp_ref)
    agg = _unpack(aggp_ref[0] + aggp_ref[1] + t2_ref[...])
    h2 = jnp.maximum(agg * dinv + b2_ref[...], 0.0)
    rowid = lax.broadcasted_iota(jnp.int32, (BR, 1), 0) + i * BR
    h2 = jnp.where(rowid < N_NODES, h2, 0.0)
    gids = lax.broadcasted_iota(jnp.int32, (NUM_GRAPHS, BR), 0)
    colid = lax.broadcasted_iota(jnp.int32, (NUM_GRAPHS, BR), 1) + i * BR
    onehot = ((batch_ref[...] == gids) & (colid < N_NODES)).astype(jnp.float32)
    part = jnp.dot(onehot, h2, preferred_element_type=jnp.float32,
                   precision=lax.Precision.HIGHEST)
    pcnt = jnp.sum(onehot, axis=1, keepdims=True)

    @pl.when(i == 0)
    def _():
        sums_ref[...] = part
        counts_ref[...] = pcnt

    @pl.when(i > 0)
    def _():
        sums_ref[...] += part
        counts_ref[...] += pcnt

    @pl.when(i == NBLK - 1)
    def _():
        pooled = sums_ref[...] / jnp.maximum(counts_ref[...], 1.0)
        out_ref[...] = jnp.dot(pooled, fcw_ref[...],
                               preferred_element_type=jnp.float32) \
            + fcb_ref[...]


def _row_spec(width):
    return pl.BlockSpec((BR, width), lambda i: (i, 0))


def _full_spec(shape):
    nd = len(shape)
    return pl.BlockSpec(shape, lambda i, _nd=nd: (0,) * _nd)


_DEGP_SPEC = pl.BlockSpec((NC, BR // 2, 32), lambda i: (0, i, 0))
_AGGP_SPEC = pl.BlockSpec((NC, BR // 2, 2 * HIDDEN), lambda i: (0, i, 0))
_PACK_SPEC = pl.BlockSpec((BR // 2, 2 * HIDDEN), lambda i: (i, 0))


def _dense1(x, W1, degp):
    return pl.pallas_call(
        _dense1_body,
        grid=(NBLK,),
        in_specs=[_row_spec(D_FEAT), _full_spec((D_FEAT, HIDDEN)),
                  _DEGP_SPEC],
        out_specs=_PACK_SPEC,
        out_shape=jax.ShapeDtypeStruct((NPAD // 2, 2 * HIDDEN), jnp.float32),
    )(x, W1, degp)


def _dense2(aggp, t1, degp, b1, W2):
    return pl.pallas_call(
        _dense2_body,
        grid=(NBLK,),
        in_specs=[_AGGP_SPEC, _PACK_SPEC, _DEGP_SPEC,
                  _full_spec((1, HIDDEN)), _full_spec((HIDDEN, HIDDEN))],
        out_specs=_PACK_SPEC,
        out_shape=jax.ShapeDtypeStruct((NPAD // 2, 2 * HIDDEN), jnp.float32),
    )(aggp, t1, degp, b1, W2)


def _dense3(aggp, t2, degp, b2, batch2d, fc_w, fc_b):
    return pl.pallas_call(
        _dense3_body,
        grid=(NBLK,),
        in_specs=[_AGGP_SPEC, _PACK_SPEC, _DEGP_SPEC,
                  _full_spec((1, HIDDEN)),
                  pl.BlockSpec((1, BR), lambda i: (0, i)),
                  _full_spec((HIDDEN, 1)), _full_spec((1, 1))],
        out_specs=_full_spec((NUM_GRAPHS, 1)),
        out_shape=jax.ShapeDtypeStruct((NUM_GRAPHS, 1), jnp.float32),
        scratch_shapes=[pltpu.VMEM((NUM_GRAPHS, HIDDEN), jnp.float32),
                        pltpu.VMEM((NUM_GRAPHS, 1), jnp.float32)],
    )(aggp, t2, degp, b2, batch2d, fc_w, fc_b)


def _pos(v):
    rem = v % BR
    return (v - rem) + 2 * (rem % (BR // 2)) + rem // (BR // 2)


def kernel(x, edge_index, batch, W1, b1, W2, b2, fc_w, fc_b):
    src = _pos(edge_index[0].astype(jnp.int32)).reshape(NW, NCHUNK, CHUNK)
    dst = _pos(edge_index[1].astype(jnp.int32)).reshape(NW, NCHUNK, CHUNK)
    batch2d = batch.astype(jnp.int32).reshape(1, N_NODES)

    ones16 = jnp.ones((CHUNK, 16), jnp.float32)
    zeros16 = jnp.zeros((NPAD, 16), jnp.float32)
    zeros64 = jnp.zeros((NPAD, HIDDEN), jnp.float32)

    degp = _deg_kernel(dst, ones16, zeros16)
    degp32 = degp.reshape(NC, NPAD // 2, 32)

    t1p = _dense1(x, W1, degp32)
    agg1 = _agg_kernel(t1p.reshape(NPAD, HIDDEN), src, dst, zeros64)
    t2p = _dense2(agg1.reshape(NC, NPAD // 2, 2 * HIDDEN), t1p, degp32,
                  b1.reshape(1, HIDDEN), W2)
    agg2 = _agg_kernel(t2p.reshape(NPAD, HIDDEN), src, dst, zeros64)
    out = _dense3(agg2.reshape(NC, NPAD // 2, 2 * HIDDEN), t2p, degp32,
                  b2.reshape(1, HIDDEN), batch2d, fc_w, fc_b.reshape(1, 1))
    return out
